# baseline (speedup 1.0000x reference)
import functools

import jax
import jax.numpy as jnp
from jax import lax
from jax.experimental import pallas as pl
from jax.experimental.pallas import tpu as pltpu
from jax.experimental.pallas import tpu_sc as plsc

DIM = 64
MARGIN = 1.0
BATCH = 16384
CHUNK = 32
LANES = 16


def _rsqrt(x):
    i = plsc.bitcast(x, jnp.int32)
    y = plsc.bitcast(jnp.int32(0x5F3759DF) - (i >> 1), jnp.float32)
    for _ in range(3):
        y = y * (1.5 - 0.5 * x * y * y)
    return y


def _make_sc_kernel(num_workers, bpw):
    mesh = plsc.VectorSubcoreMesh(core_axis_name="c", subcore_axis_name="s")
    num_cores = mesh.num_cores
    nchunk = bpw // CHUNK

    @functools.partial(
        pl.kernel,
        mesh=mesh,
        compiler_params=pltpu.CompilerParams(needs_layout_passes=False,
                                             skip_device_barrier=True),
        out_type=jax.ShapeDtypeStruct((num_workers, 128), jnp.float32),
        scratch_types=[
            pltpu.VMEM((bpw,), jnp.int32),
            pltpu.VMEM((bpw,), jnp.int32),
            pltpu.VMEM((bpw,), jnp.int32),
            pltpu.VMEM((CHUNK, DIM), jnp.float32),
            pltpu.VMEM((CHUNK, DIM), jnp.float32),
            pltpu.VMEM((CHUNK, DIM), jnp.float32),
            pltpu.VMEM((128,), jnp.float32),
            pltpu.SemaphoreType.DMA,
        ],
    )
    def trans_e_cost(lidx_hbm, ridx_hbm, qidx_hbm, ent_hbm, rel_hbm,
                     out_hbm, lv, rv, qv,
                     lrows, rrows, qrows, outv, sem):
        wid = lax.axis_index("s") * num_cores + lax.axis_index("c")
        base = wid * bpw
        pltpu.sync_copy(lidx_hbm.at[pl.ds(base, bpw)], lv)
        pltpu.sync_copy(ridx_hbm.at[pl.ds(base, bpw)], rv)
        pltpu.sync_copy(qidx_hbm.at[pl.ds(base, bpw)], qv)

        iota = lax.iota(jnp.int32, LANES)
        zero = jnp.zeros((LANES,), jnp.float32)

        def chunk_body(ch, acc):
            off = ch * CHUNK
            copies = []
            for k in range(CHUNK // LANES):
                lidx = lv[pl.ds(off + k * LANES, LANES)]
                ridx = rv[pl.ds(off + k * LANES, LANES)]
                qidx = qv[pl.ds(off + k * LANES, LANES)]
                for j in range(LANES):
                    kk = k * LANES + j
                    copies.append(pltpu.async_copy(
                        ent_hbm.at[lidx[j]], lrows.at[kk], sem))
                    copies.append(pltpu.async_copy(
                        ent_hbm.at[ridx[j]], rrows.at[kk], sem))
                    copies.append(pltpu.async_copy(
                        rel_hbm.at[qidx[j]], qrows.at[kk], sem))
            for cpy in copies:
                cpy.wait()
            for g in range(CHUNK // LANES):
                rowloc = g * LANES + iota
                sl = sr = sq = dlr = dqr = zero
                for c in range(DIM):
                    ci = jnp.full((LANES,), c, jnp.int32)
                    lc = plsc.load_gather(lrows, [rowloc, ci])
                    rc = plsc.load_gather(rrows, [rowloc, ci])
                    qc = plsc.load_gather(qrows, [rowloc, ci])
                    sl = sl + lc * lc
                    sr = sr + rc * rc
                    sq = sq + qc * qc
                    dlr = dlr + lc * rc
                    dqr = dqr + qc * rc
                tiny = jnp.float32(1e-24)
                simi = (dlr * _rsqrt(jnp.maximum(sl * sr, tiny))
                        + dqr * _rsqrt(jnp.maximum(sq * sr, tiny)))
                similn = simi
                simirn = simi
                costl = jnp.maximum(similn - simi + MARGIN, 0.0)
                costr = jnp.maximum(simirn - simi + MARGIN, 0.0)
                acc = acc + costl + costr
            return acc

        acc = lax.fori_loop(0, nchunk, chunk_body, zero)
        total = jnp.sum(acc) * jnp.float32(1.0 / BATCH)
        outlane = jnp.where(iota == 0, total, 0.0)
        for k in range(128 // LANES):
            outv[pl.ds(k * LANES, LANES)] = outlane if k == 0 else zero
        pltpu.sync_copy(outv, out_hbm.at[wid])

    return trans_e_cost


def kernel(leftEnIndices, rightEnIndices, relIndices, negLeftEnIndices,
           negRightEnIndices, entityEmbedding, relationEmbedding):
    del negLeftEnIndices, negRightEnIndices
    info = plsc.get_sparse_core_info()
    num_workers = info.num_cores * info.num_subcores
    bpw = BATCH // num_workers
    sc = _make_sc_kernel(num_workers, bpw)
    partials = sc(leftEnIndices.astype(jnp.int32),
                  rightEnIndices.astype(jnp.int32),
                  relIndices.astype(jnp.int32),
                  entityEmbedding, relationEmbedding)
    return jnp.sum(partials)


# speedup vs baseline: 1.3330x; 1.3330x over previous
import functools

import jax
import jax.numpy as jnp
from jax import lax
from jax.experimental import pallas as pl
from jax.experimental.pallas import tpu as pltpu
from jax.experimental.pallas import tpu_sc as plsc

DIM = 64
MARGIN = 1.0
BATCH = 16384
CHUNK = 16
LANES = 16


def _rsqrt(x):
    i = plsc.bitcast(x, jnp.int32)
    y = plsc.bitcast(jnp.int32(0x5F3759DF) - (i >> 1), jnp.float32)
    for _ in range(3):
        y = y * (1.5 - 0.5 * x * y * y)
    return y


def _make_sc_kernel(num_workers, bpw):
    mesh = plsc.VectorSubcoreMesh(core_axis_name="c", subcore_axis_name="s")
    num_cores = mesh.num_cores
    nchunk = bpw // CHUNK

    tilebuf = pltpu.VMEM((CHUNK, 8, DIM), jnp.float32)

    @functools.partial(
        pl.kernel,
        mesh=mesh,
        compiler_params=pltpu.CompilerParams(needs_layout_passes=False),
        out_type=jax.ShapeDtypeStruct((num_workers, 128), jnp.float32),
        scratch_types=[
            pltpu.VMEM((bpw,), jnp.int32),
            pltpu.VMEM((bpw,), jnp.int32),
            pltpu.VMEM((bpw,), jnp.int32),
            tilebuf, tilebuf, tilebuf,
            tilebuf, tilebuf, tilebuf,
            pltpu.VMEM((128,), jnp.float32),
            pltpu.SemaphoreType.DMA,
            pltpu.SemaphoreType.DMA,
        ],
    )
    def trans_e_cost(lidx_hbm, ridx_hbm, qidx_hbm, ent_hbm, rel_hbm,
                     out_hbm, lv, rv, qv, la, ra, qa, lb, rb, qb,
                     outv, sema, semb):
        wid = lax.axis_index("s") * num_cores + lax.axis_index("c")
        base = wid * bpw
        pltpu.sync_copy(lidx_hbm.at[pl.ds(base, bpw)], lv)
        pltpu.sync_copy(ridx_hbm.at[pl.ds(base, bpw)], rv)
        pltpu.sync_copy(qidx_hbm.at[pl.ds(base, bpw)], qv)

        iota = lax.iota(jnp.int32, LANES)
        zero = jnp.zeros((LANES,), jnp.float32)

        def fire(ch, lt, rt, qt, sem):
            off = ch * CHUNK
            lslab = lv[pl.ds(off, LANES)] >> 3
            rslab = rv[pl.ds(off, LANES)] >> 3
            qslab = qv[pl.ds(off, LANES)] >> 3
            for j in range(LANES):
                pltpu.async_copy(ent_hbm.at[lslab[j]], lt.at[j], sem)
                pltpu.async_copy(ent_hbm.at[rslab[j]], rt.at[j], sem)
                pltpu.async_copy(rel_hbm.at[qslab[j]], qt.at[j], sem)

        def drain(lt, rt, qt, sem):
            pltpu.make_async_copy(ent_hbm.at[pl.ds(0, CHUNK)], lt, sem).wait()
            pltpu.make_async_copy(ent_hbm.at[pl.ds(0, CHUNK)], rt, sem).wait()
            pltpu.make_async_copy(rel_hbm.at[pl.ds(0, CHUNK)], qt, sem).wait()

        def compute(ch, lt, rt, qt, acc):
            off = ch * CHUNK
            lsub = lv[pl.ds(off, LANES)] & 7
            rsub = rv[pl.ds(off, LANES)] & 7
            qsub = qv[pl.ds(off, LANES)] & 7
            sl = sr = sq = dlr = dqr = zero
            for c in range(DIM):
                ci = jnp.full((LANES,), c, jnp.int32)
                lc = plsc.load_gather(lt, [iota, lsub, ci])
                rc = plsc.load_gather(rt, [iota, rsub, ci])
                qc = plsc.load_gather(qt, [iota, qsub, ci])
                sl = sl + lc * lc
                sr = sr + rc * rc
                sq = sq + qc * qc
                dlr = dlr + lc * rc
                dqr = dqr + qc * rc
            tiny = jnp.float32(1e-24)
            simi = (dlr * _rsqrt(jnp.maximum(sl * sr, tiny))
                    + dqr * _rsqrt(jnp.maximum(sq * sr, tiny)))
            similn = simi
            simirn = simi
            costl = jnp.maximum(similn - simi + MARGIN, 0.0)
            costr = jnp.maximum(simirn - simi + MARGIN, 0.0)
            return acc + costl + costr

        fire(0, la, ra, qa, sema)

        def pair_body(p, acc):
            ch0 = p * 2
            fire(ch0 + 1, lb, rb, qb, semb)
            drain(la, ra, qa, sema)
            acc = compute(ch0, la, ra, qa, acc)

            @pl.when(ch0 + 2 < nchunk)
            def _():
                fire(ch0 + 2, la, ra, qa, sema)

            drain(lb, rb, qb, semb)
            acc = compute(ch0 + 1, lb, rb, qb, acc)
            return acc

        acc = lax.fori_loop(0, nchunk // 2, pair_body, zero)
        total = jnp.sum(acc) * jnp.float32(1.0 / BATCH)
        outlane = jnp.where(iota == 0, total, 0.0)
        for k in range(128 // LANES):
            outv[pl.ds(k * LANES, LANES)] = outlane if k == 0 else zero
        pltpu.sync_copy(outv, out_hbm.at[wid])

    return trans_e_cost


def kernel(leftEnIndices, rightEnIndices, relIndices, negLeftEnIndices,
           negRightEnIndices, entityEmbedding, relationEmbedding):
    del negLeftEnIndices, negRightEnIndices
    info = plsc.get_sparse_core_info()
    num_workers = info.num_cores * info.num_subcores
    bpw = BATCH // num_workers
    nent, dim = entityEmbedding.shape
    nrel = relationEmbedding.shape[0]
    ent3 = jnp.reshape(entityEmbedding, (nent // 8, 8, dim))
    rel3 = jnp.reshape(relationEmbedding, (nrel // 8, 8, dim))
    sc = _make_sc_kernel(num_workers, bpw)
    partials = sc(leftEnIndices.astype(jnp.int32),
                  rightEnIndices.astype(jnp.int32),
                  relIndices.astype(jnp.int32), ent3, rel3)
    return jnp.sum(partials)
